# baseline (speedup 1.0000x reference)
import jax
import jax.numpy as jnp
from jax import lax
from jax.experimental import pallas as pl

_ALPHA = 0.5
_N_BLOCK = 1024


def _linear_kernel(x_ref, w_in_ref, b_in_ref, w_out_ref, b_out_ref, o_ref):
    w = _ALPHA * w_in_ref[...] + (1.0 - _ALPHA) * w_out_ref[...]
    b = _ALPHA * b_in_ref[...] + (1.0 - _ALPHA) * b_out_ref[...]
    acc = lax.dot_general(
        w, x_ref[0],
        dimension_numbers=(((1,), (1,)), ((), ())),
        preferred_element_type=jnp.float32,
    )
    o_ref[0] = acc + b[:, None]


def kernel(x, At, W_in, b_in, W_out, b_out):
    del At
    Bd, Nd, L = x.shape
    out_ch = W_in.shape[0]

    grid = (Bd, Nd // _N_BLOCK)
    out_t = pl.pallas_call(
        _linear_kernel,
        grid=grid,
        in_specs=[
            pl.BlockSpec((1, _N_BLOCK, L), lambda bi, j: (bi, j, 0)),
            pl.BlockSpec((out_ch, L), lambda bi, j: (0, 0)),
            pl.BlockSpec((out_ch,), lambda bi, j: (0,)),
            pl.BlockSpec((out_ch, L), lambda bi, j: (0, 0)),
            pl.BlockSpec((out_ch,), lambda bi, j: (0,)),
        ],
        out_specs=pl.BlockSpec((1, out_ch, _N_BLOCK), lambda bi, j: (bi, 0, j)),
        out_shape=jax.ShapeDtypeStruct((Bd, out_ch, Nd), jnp.float32),
    )(x, W_in, b_in, W_out, b_out)
    return out_t.transpose(0, 2, 1)


# speedup vs baseline: 1.6865x; 1.6865x over previous
import jax
import jax.numpy as jnp
from jax import lax
from jax.experimental import pallas as pl

_ALPHA = 0.5
_B_BLOCK = 2


def _linear_kernel(x_ref, w_in_ref, b_in_ref, w_out_ref, b_out_ref, o_ref):
    w = _ALPHA * w_in_ref[...] + (1.0 - _ALPHA) * w_out_ref[...]
    b = _ALPHA * b_in_ref[...] + (1.0 - _ALPHA) * b_out_ref[...]
    acc = lax.dot_general(
        w, x_ref[...],
        dimension_numbers=(((1,), (2,)), ((), ())),
        preferred_element_type=jnp.float32,
    )
    o_ref[...] = acc.transpose(1, 0, 2) + b[None, :, None]


def kernel(x, At, W_in, b_in, W_out, b_out):
    del At
    Bd, Nd, L = x.shape
    out_ch = W_in.shape[0]

    grid = (Bd // _B_BLOCK,)
    out_t = pl.pallas_call(
        _linear_kernel,
        grid=grid,
        in_specs=[
            pl.BlockSpec((_B_BLOCK, Nd, L), lambda i: (i, 0, 0)),
            pl.BlockSpec((out_ch, L), lambda i: (0, 0)),
            pl.BlockSpec((out_ch,), lambda i: (0,)),
            pl.BlockSpec((out_ch, L), lambda i: (0, 0)),
            pl.BlockSpec((out_ch,), lambda i: (0,)),
        ],
        out_specs=pl.BlockSpec((_B_BLOCK, out_ch, Nd), lambda i: (i, 0, 0)),
        out_shape=jax.ShapeDtypeStruct((Bd, out_ch, Nd), jnp.float32),
    )(x, W_in, b_in, W_out, b_out)
    return out_t.transpose(0, 2, 1)


# speedup vs baseline: 1.9000x; 1.1266x over previous
import jax
import jax.numpy as jnp
from jax import lax
from jax.experimental import pallas as pl

_ALPHA = 0.5
_B_BLOCK = 2


def _linear_kernel(x_ref, w_in_ref, b_in_ref, w_out_ref, b_out_ref, o_ref):
    w = _ALPHA * w_in_ref[...] + (1.0 - _ALPHA) * w_out_ref[...]
    b = _ALPHA * b_in_ref[...] + (1.0 - _ALPHA) * b_out_ref[...]
    bcol = b[:, None]
    for bb in range(_B_BLOCK):
        acc = lax.dot_general(
            w, x_ref[bb],
            dimension_numbers=(((1,), (1,)), ((), ())),
            preferred_element_type=jnp.float32,
        )
        o_ref[bb] = acc + bcol


def kernel(x, At, W_in, b_in, W_out, b_out):
    del At
    Bd, Nd, L = x.shape
    out_ch = W_in.shape[0]

    grid = (Bd // _B_BLOCK,)
    out_t = pl.pallas_call(
        _linear_kernel,
        grid=grid,
        in_specs=[
            pl.BlockSpec((_B_BLOCK, Nd, L), lambda i: (i, 0, 0)),
            pl.BlockSpec((out_ch, L), lambda i: (0, 0)),
            pl.BlockSpec((out_ch,), lambda i: (0,)),
            pl.BlockSpec((out_ch, L), lambda i: (0, 0)),
            pl.BlockSpec((out_ch,), lambda i: (0,)),
        ],
        out_specs=pl.BlockSpec((_B_BLOCK, out_ch, Nd), lambda i: (i, 0, 0)),
        out_shape=jax.ShapeDtypeStruct((Bd, out_ch, Nd), jnp.float32),
    )(x, W_in, b_in, W_out, b_out)
    return out_t.transpose(0, 2, 1)
